# baseline (speedup 1.0000x reference)
import dataclasses
import functools

import jax
import jax.numpy as jnp
import numpy as np
from jax import lax
from jax.experimental import pallas as pl
from jax.experimental.pallas import tpu as pltpu
from jax.experimental.pallas import tpu_sc as plsc

DIM = 128
BATCH = 16384
NC = 2
NS = 16
NW = NC * NS
BPW = BATCH // NW
CHUNK = 128
NCHUNK = BPW // CHUNK
NPAIR = NCHUNK // 2
LANES = 16
NSEG = DIM // LANES

_BITREV = [0, 8, 4, 12, 2, 10, 6, 14, 1, 9, 5, 13, 3, 11, 7, 15]


def kernel(center_words, context_words, center_table, context_table):
    cw = center_words.astype(jnp.int32).reshape(NW, NCHUNK, CHUNK)
    xw = context_words.astype(jnp.int32).reshape(NW, NCHUNK, CHUNK)

    mesh = plsc.VectorSubcoreMesh(core_axis_name="c", subcore_axis_name="s")

    cp = pltpu.CompilerParams()
    if "needs_layout_passes" in pltpu.CompilerParams.__dataclass_fields__:
        cp = dataclasses.replace(cp, needs_layout_passes=False)

    @functools.partial(
        pl.kernel,
        compiler_params=cp,
        out_type=jax.ShapeDtypeStruct((BATCH,), jnp.float32),
        mesh=mesh,
        scratch_types=[
            pltpu.VMEM((NCHUNK, CHUNK), jnp.int32),
            pltpu.VMEM((NCHUNK, CHUNK), jnp.int32),
            pltpu.VMEM((2, CHUNK, DIM), jnp.float32),
            pltpu.VMEM((2, CHUNK, DIM), jnp.float32),
            pltpu.VMEM((BPW,), jnp.float32),
            pltpu.VMEM((CHUNK, LANES), jnp.float32),
            pltpu.SemaphoreType.DMA,
            pltpu.SemaphoreType.DMA,
            pltpu.SemaphoreType.DMA,
            pltpu.SemaphoreType.DMA,
        ],
    )
    def skipgram(cw_hbm, xw_hbm, ct_hbm, xt_hbm, out_hbm,
                 cidx, xidx, crows, xrows, scores, accbuf,
                 sem_c0, sem_x0, sem_c1, sem_x1):
        wid = lax.axis_index("s") * NC + lax.axis_index("c")
        pltpu.sync_copy(cw_hbm.at[wid], cidx)
        pltpu.sync_copy(xw_hbm.at[wid], xidx)

        lane = lax.iota(jnp.int32, LANES)
        perms = {k: (lane + k) & (LANES - 1) for k in (1, 2, 4, 8, 12, 14, 15)}
        masks = {s: (lane & (s - 1)) < (s // 2) for s in (16, 8, 4, 2)}

        gather_dnums = lax.GatherDimensionNumbers(
            offset_dims=(), collapsed_slice_dims=(0,), start_index_map=(0,))

        def rot(x, k):
            return lax.gather(
                x, perms[k % LANES][:, None], gather_dnums, (1,),
                mode=lax.GatherScatterMode.PROMISE_IN_BOUNDS)

        def merge(a, b, s):
            k = s // 2
            return jnp.where(masks[s], a + rot(a, k), b + rot(b, -k))

        sems = ((sem_c0, sem_x0), (sem_c1, sem_x1))

        HC = CHUNK // 2

        def issue(j, slot):
            for h in (0, 1):
                pltpu.async_copy(
                    ct_hbm.at[cidx.at[j, pl.ds(h * HC, HC)]],
                    crows.at[slot, pl.ds(h * HC, HC)], sems[slot][0])
                pltpu.async_copy(
                    xt_hbm.at[xidx.at[j, pl.ds(h * HC, HC)]],
                    xrows.at[slot, pl.ds(h * HC, HC)], sems[slot][1])

        def wait(slot):
            for h in (0, 1):
                pltpu.make_async_copy(
                    ct_hbm.at[cidx.at[0, pl.ds(h * HC, HC)]],
                    crows.at[slot, pl.ds(h * HC, HC)], sems[slot][0]).wait()
                pltpu.make_async_copy(
                    xt_hbm.at[xidx.at[0, pl.ds(h * HC, HC)]],
                    xrows.at[slot, pl.ds(h * HC, HC)], sems[slot][1]).wait()

        def compute(j, slot):
            @plsc.parallel_loop(0, CHUNK)
            def _(r):
                acc = (crows[slot, r, pl.ds(0, LANES)]
                       * xrows[slot, r, pl.ds(0, LANES)])
                for t in range(1, NSEG):
                    acc = acc + (crows[slot, r, pl.ds(t * LANES, LANES)]
                                 * xrows[slot, r, pl.ds(t * LANES, LANES)])
                accbuf[r, pl.ds(0, LANES)] = acc

            @plsc.parallel_loop(0, CHUNK // LANES)
            def _(g):
                stack = []
                for leaf in range(LANES):
                    r = g * LANES + _BITREV[leaf]
                    node = (16, accbuf[r, pl.ds(0, LANES)])
                    while stack and stack[-1][0] == node[0]:
                        s, a = stack.pop()
                        node = (s // 2, merge(a, node[1], s))
                    stack.append(node)
                scores[pl.ds(j * CHUNK + g * LANES, LANES)] = stack[0][1]

        issue(0, 0)
        issue(1, 1)

        @pl.loop(0, NPAIR)
        def _(p):
            for sl in (0, 1):
                jj = 2 * p + sl
                wait(sl)
                compute(jj, sl)
                nxt = jj + 2

                @pl.when(nxt < NCHUNK)
                def _():
                    issue(nxt, sl)

        pltpu.sync_copy(scores, out_hbm.at[pl.ds(wid * BPW, BPW)])

    return skipgram(cw, xw, center_table, context_table)


# speedup vs baseline: 1.0293x; 1.0293x over previous
import dataclasses
import functools

import jax
import jax.numpy as jnp
import numpy as np
from jax import lax
from jax.experimental import pallas as pl
from jax.experimental.pallas import tpu as pltpu
from jax.experimental.pallas import tpu_sc as plsc

DIM = 128
BATCH = 16384
NC = 2
NS = 16
NW = NC * NS
BPW = BATCH // NW
CHUNK = 64
NCHUNK = BPW // CHUNK
SLOTS = 4
NQUAD = NCHUNK // SLOTS
LANES = 16
NSEG = DIM // LANES

_BITREV = [0, 8, 4, 12, 2, 10, 6, 14, 1, 9, 5, 13, 3, 11, 7, 15]


def kernel(center_words, context_words, center_table, context_table):
    cw = center_words.astype(jnp.int32).reshape(NW, NCHUNK, CHUNK)
    xw = context_words.astype(jnp.int32).reshape(NW, NCHUNK, CHUNK)

    mesh = plsc.VectorSubcoreMesh(core_axis_name="c", subcore_axis_name="s")

    cp = pltpu.CompilerParams()
    if "needs_layout_passes" in pltpu.CompilerParams.__dataclass_fields__:
        cp = dataclasses.replace(cp, needs_layout_passes=False)

    @functools.partial(
        pl.kernel,
        compiler_params=cp,
        out_type=jax.ShapeDtypeStruct((BATCH,), jnp.float32),
        mesh=mesh,
        scratch_types=[
            pltpu.VMEM((NCHUNK, CHUNK), jnp.int32),
            pltpu.VMEM((NCHUNK, CHUNK), jnp.int32),
            pltpu.VMEM((SLOTS, CHUNK, DIM), jnp.float32),
            pltpu.VMEM((SLOTS, CHUNK, DIM), jnp.float32),
            pltpu.VMEM((BPW,), jnp.float32),
            pltpu.VMEM((CHUNK, LANES), jnp.float32),
        ] + [pltpu.SemaphoreType.DMA] * (2 * SLOTS),
    )
    def skipgram(cw_hbm, xw_hbm, ct_hbm, xt_hbm, out_hbm,
                 cidx, xidx, crows, xrows, scores, accbuf, *dma_sems):
        wid = lax.axis_index("s") * NC + lax.axis_index("c")
        pltpu.sync_copy(cw_hbm.at[wid], cidx)
        pltpu.sync_copy(xw_hbm.at[wid], xidx)

        lane = lax.iota(jnp.int32, LANES)
        perms = {k: (lane + k) & (LANES - 1) for k in (1, 2, 4, 8, 12, 14, 15)}
        masks = {s: (lane & (s - 1)) < (s // 2) for s in (16, 8, 4, 2)}

        gather_dnums = lax.GatherDimensionNumbers(
            offset_dims=(), collapsed_slice_dims=(0,), start_index_map=(0,))

        def rot(x, k):
            return lax.gather(
                x, perms[k % LANES][:, None], gather_dnums, (1,),
                mode=lax.GatherScatterMode.PROMISE_IN_BOUNDS)

        def merge(a, b, s):
            k = s // 2
            return jnp.where(masks[s], a + rot(a, k), b + rot(b, -k))

        sems = tuple((dma_sems[2 * i], dma_sems[2 * i + 1])
                     for i in range(SLOTS))

        def issue(j, slot):
            pltpu.async_copy(ct_hbm.at[cidx.at[j]], crows.at[slot],
                             sems[slot][0])
            pltpu.async_copy(xt_hbm.at[xidx.at[j]], xrows.at[slot],
                             sems[slot][1])

        def wait(slot):
            pltpu.make_async_copy(ct_hbm.at[cidx.at[0]], crows.at[slot],
                                  sems[slot][0]).wait()
            pltpu.make_async_copy(xt_hbm.at[xidx.at[0]], xrows.at[slot],
                                  sems[slot][1]).wait()

        def compute(j, slot):
            @plsc.parallel_loop(0, CHUNK)
            def _(r):
                acc = (crows[slot, r, pl.ds(0, LANES)]
                       * xrows[slot, r, pl.ds(0, LANES)])
                for t in range(1, NSEG):
                    acc = acc + (crows[slot, r, pl.ds(t * LANES, LANES)]
                                 * xrows[slot, r, pl.ds(t * LANES, LANES)])
                accbuf[r, pl.ds(0, LANES)] = acc

            @plsc.parallel_loop(0, CHUNK // LANES)
            def _(g):
                stack = []
                for leaf in range(LANES):
                    r = g * LANES + _BITREV[leaf]
                    node = (16, accbuf[r, pl.ds(0, LANES)])
                    while stack and stack[-1][0] == node[0]:
                        s, a = stack.pop()
                        node = (s // 2, merge(a, node[1], s))
                    stack.append(node)
                scores[pl.ds(j * CHUNK + g * LANES, LANES)] = stack[0][1]

        for s0 in range(SLOTS):
            issue(s0, s0)

        @pl.loop(0, NQUAD)
        def _(p):
            for sl in range(SLOTS):
                jj = SLOTS * p + sl
                wait(sl)
                compute(jj, sl)
                nxt = jj + SLOTS

                @pl.when(nxt < NCHUNK)
                def _():
                    issue(nxt, sl)

        pltpu.sync_copy(scores, out_hbm.at[pl.ds(wid * BPW, BPW)])

    return skipgram(cw, xw, center_table, context_table)


# speedup vs baseline: 1.0304x; 1.0011x over previous
import dataclasses
import functools

import jax
import jax.numpy as jnp
import numpy as np
from jax import lax
from jax.experimental import pallas as pl
from jax.experimental.pallas import tpu as pltpu
from jax.experimental.pallas import tpu_sc as plsc

DIM = 128
BATCH = 16384
NC = 2
NS = 16
NW = NC * NS
BPW = BATCH // NW
CHUNK = 64
NCHUNK = BPW // CHUNK
SLOTS = 4
NQUAD = NCHUNK // SLOTS
LANES = 16
NSEG = DIM // LANES

_BITREV = [0, 8, 4, 12, 2, 10, 6, 14, 1, 9, 5, 13, 3, 11, 7, 15]


def kernel(center_words, context_words, center_table, context_table):
    cw = center_words.astype(jnp.int32)
    xw = context_words.astype(jnp.int32)

    mesh = plsc.VectorSubcoreMesh(core_axis_name="c", subcore_axis_name="s")

    cp = pltpu.CompilerParams()
    if "needs_layout_passes" in pltpu.CompilerParams.__dataclass_fields__:
        cp = dataclasses.replace(cp, needs_layout_passes=False)

    @functools.partial(
        pl.kernel,
        compiler_params=cp,
        out_type=jax.ShapeDtypeStruct((BATCH,), jnp.float32),
        mesh=mesh,
        scratch_types=[
            pltpu.VMEM((BPW,), jnp.int32),
            pltpu.VMEM((BPW,), jnp.int32),
            pltpu.VMEM((SLOTS, CHUNK, DIM), jnp.float32),
            pltpu.VMEM((SLOTS, CHUNK, DIM), jnp.float32),
            pltpu.VMEM((BPW,), jnp.float32),
            pltpu.VMEM((CHUNK, LANES), jnp.float32),
        ] + [pltpu.SemaphoreType.DMA] * (2 * SLOTS),
    )
    def skipgram(cw_hbm, xw_hbm, ct_hbm, xt_hbm, out_hbm,
                 cidx, xidx, crows, xrows, scores, accbuf, *dma_sems):
        wid = lax.axis_index("s") * NC + lax.axis_index("c")
        pltpu.sync_copy(cw_hbm.at[pl.ds(wid * BPW, BPW)], cidx)
        pltpu.sync_copy(xw_hbm.at[pl.ds(wid * BPW, BPW)], xidx)

        lane = lax.iota(jnp.int32, LANES)
        perms = {k: (lane + k) & (LANES - 1) for k in (1, 2, 4, 8, 12, 14, 15)}
        masks = {s: (lane & (s - 1)) < (s // 2) for s in (16, 8, 4, 2)}

        gather_dnums = lax.GatherDimensionNumbers(
            offset_dims=(), collapsed_slice_dims=(0,), start_index_map=(0,))

        def rot(x, k):
            return lax.gather(
                x, perms[k % LANES][:, None], gather_dnums, (1,),
                mode=lax.GatherScatterMode.PROMISE_IN_BOUNDS)

        def merge(a, b, s):
            k = s // 2
            return jnp.where(masks[s], a + rot(a, k), b + rot(b, -k))

        sems = tuple((dma_sems[2 * i], dma_sems[2 * i + 1])
                     for i in range(SLOTS))

        def issue(j, slot):
            sel = pl.ds(j * CHUNK, CHUNK)
            pltpu.async_copy(ct_hbm.at[cidx.at[sel]], crows.at[slot],
                             sems[slot][0])
            pltpu.async_copy(xt_hbm.at[xidx.at[sel]], xrows.at[slot],
                             sems[slot][1])

        def wait(slot):
            sel = pl.ds(0, CHUNK)
            pltpu.make_async_copy(ct_hbm.at[cidx.at[sel]], crows.at[slot],
                                  sems[slot][0]).wait()
            pltpu.make_async_copy(xt_hbm.at[xidx.at[sel]], xrows.at[slot],
                                  sems[slot][1]).wait()

        def compute(j, slot):
            @plsc.parallel_loop(0, CHUNK)
            def _(r):
                acc = (crows[slot, r, pl.ds(0, LANES)]
                       * xrows[slot, r, pl.ds(0, LANES)])
                for t in range(1, NSEG):
                    acc = acc + (crows[slot, r, pl.ds(t * LANES, LANES)]
                                 * xrows[slot, r, pl.ds(t * LANES, LANES)])
                accbuf[r, pl.ds(0, LANES)] = acc

            @plsc.parallel_loop(0, CHUNK // LANES)
            def _(g):
                stack = []
                for leaf in range(LANES):
                    r = g * LANES + _BITREV[leaf]
                    node = (16, accbuf[r, pl.ds(0, LANES)])
                    while stack and stack[-1][0] == node[0]:
                        s, a = stack.pop()
                        node = (s // 2, merge(a, node[1], s))
                    stack.append(node)
                scores[pl.ds(j * CHUNK + g * LANES, LANES)] = stack[0][1]

        for s0 in range(SLOTS):
            issue(s0, s0)

        @pl.loop(0, NQUAD)
        def _(p):
            for sl in range(SLOTS):
                jj = SLOTS * p + sl
                wait(sl)
                compute(jj, sl)
                nxt = jj + SLOTS

                @pl.when(nxt < NCHUNK)
                def _():
                    issue(nxt, sl)

        pltpu.sync_copy(scores, out_hbm.at[pl.ds(wid * BPW, BPW)])

    return skipgram(cw, xw, center_table, context_table)


# speedup vs baseline: 1.0606x; 1.0293x over previous
import dataclasses
import functools

import jax
import jax.numpy as jnp
import numpy as np
from jax import lax
from jax.experimental import pallas as pl
from jax.experimental.pallas import tpu as pltpu
from jax.experimental.pallas import tpu_sc as plsc

DIM = 128
BATCH = 16384
NC = 2
NS = 16
NW = NC * NS
BPW = BATCH // NW
CHUNK = 64
NCHUNK = BPW // CHUNK
SLOTS = 4
NQUAD = NCHUNK // SLOTS
LANES = 16
NSEG = DIM // LANES

_BITREV = [0, 8, 4, 12, 2, 10, 6, 14, 1, 9, 5, 13, 3, 11, 7, 15]


def kernel(center_words, context_words, center_table, context_table):
    cw = center_words.astype(jnp.int32)
    xw = context_words.astype(jnp.int32)

    mesh = plsc.VectorSubcoreMesh(core_axis_name="c", subcore_axis_name="s")

    cp = pltpu.CompilerParams()
    if "needs_layout_passes" in pltpu.CompilerParams.__dataclass_fields__:
        cp = dataclasses.replace(cp, needs_layout_passes=False)

    @functools.partial(
        pl.kernel,
        compiler_params=cp,
        out_type=jax.ShapeDtypeStruct((BATCH,), jnp.float32),
        mesh=mesh,
        scratch_types=[
            pltpu.VMEM((BPW,), jnp.int32),
            pltpu.VMEM((BPW,), jnp.int32),
            pltpu.VMEM((SLOTS, CHUNK, DIM), jnp.float32),
            pltpu.VMEM((SLOTS, CHUNK, DIM), jnp.float32),
            pltpu.VMEM((BPW,), jnp.float32),
            pltpu.VMEM((CHUNK, LANES), jnp.float32),
        ] + [pltpu.SemaphoreType.DMA] * (2 * SLOTS),
    )
    def skipgram(cw_hbm, xw_hbm, ct_hbm, xt_hbm, out_hbm,
                 cidx, xidx, crows, xrows, scores, accbuf, *dma_sems):
        wid = lax.axis_index("s") * NC + lax.axis_index("c")
        base = wid * BPW
        c0 = pltpu.make_async_copy(cw_hbm.at[pl.ds(base, CHUNK)],
                                   cidx.at[pl.ds(0, CHUNK)], dma_sems[0])
        x0 = pltpu.make_async_copy(xw_hbm.at[pl.ds(base, CHUNK)],
                                   xidx.at[pl.ds(0, CHUNK)], dma_sems[1])
        c0.start(); x0.start()
        REST = BPW - CHUNK
        c1 = pltpu.make_async_copy(cw_hbm.at[pl.ds(base + CHUNK, REST)],
                                   cidx.at[pl.ds(CHUNK, REST)], dma_sems[2])
        x1 = pltpu.make_async_copy(xw_hbm.at[pl.ds(base + CHUNK, REST)],
                                   xidx.at[pl.ds(CHUNK, REST)], dma_sems[3])
        c1.start(); x1.start()

        lane = lax.iota(jnp.int32, LANES)
        perms = {k: (lane + k) & (LANES - 1) for k in (1, 2, 4, 8, 12, 14, 15)}
        masks = {s: (lane & (s - 1)) < (s // 2) for s in (16, 8, 4, 2)}

        gather_dnums = lax.GatherDimensionNumbers(
            offset_dims=(), collapsed_slice_dims=(0,), start_index_map=(0,))

        def rot(x, k):
            return lax.gather(
                x, perms[k % LANES][:, None], gather_dnums, (1,),
                mode=lax.GatherScatterMode.PROMISE_IN_BOUNDS)

        def merge(a, b, s):
            k = s // 2
            return jnp.where(masks[s], a + rot(a, k), b + rot(b, -k))

        sems = tuple((dma_sems[2 * i], dma_sems[2 * i + 1])
                     for i in range(SLOTS))

        def issue(j, slot):
            sel = pl.ds(j * CHUNK, CHUNK)
            pltpu.async_copy(ct_hbm.at[cidx.at[sel]], crows.at[slot],
                             sems[slot][0])
            pltpu.async_copy(xt_hbm.at[xidx.at[sel]], xrows.at[slot],
                             sems[slot][1])

        def wait(slot):
            sel = pl.ds(0, CHUNK)
            pltpu.make_async_copy(ct_hbm.at[cidx.at[sel]], crows.at[slot],
                                  sems[slot][0]).wait()
            pltpu.make_async_copy(xt_hbm.at[xidx.at[sel]], xrows.at[slot],
                                  sems[slot][1]).wait()

        def compute(j, slot):
            @plsc.parallel_loop(0, CHUNK)
            def _(r):
                acc = (crows[slot, r, pl.ds(0, LANES)]
                       * xrows[slot, r, pl.ds(0, LANES)])
                for t in range(1, NSEG):
                    acc = acc + (crows[slot, r, pl.ds(t * LANES, LANES)]
                                 * xrows[slot, r, pl.ds(t * LANES, LANES)])
                accbuf[r, pl.ds(0, LANES)] = acc

            @plsc.parallel_loop(0, CHUNK // LANES)
            def _(g):
                stack = []
                for leaf in range(LANES):
                    r = g * LANES + _BITREV[leaf]
                    node = (16, accbuf[r, pl.ds(0, LANES)])
                    while stack and stack[-1][0] == node[0]:
                        s, a = stack.pop()
                        node = (s // 2, merge(a, node[1], s))
                    stack.append(node)
                scores[pl.ds(j * CHUNK + g * LANES, LANES)] = stack[0][1]

        c0.wait(); x0.wait()
        issue(0, 0)
        c1.wait(); x1.wait()
        for s0 in range(1, SLOTS):
            issue(s0, s0)

        @pl.loop(0, NQUAD)
        def _(p):
            for sl in range(SLOTS):
                jj = SLOTS * p + sl
                wait(sl)
                compute(jj, sl)
                nxt = jj + SLOTS

                @pl.when(nxt < NCHUNK)
                def _():
                    issue(nxt, sl)

        pltpu.sync_copy(scores, out_hbm.at[pl.ds(wid * BPW, BPW)])

    return skipgram(cw, xw, center_table, context_table)


# speedup vs baseline: 1.0693x; 1.0081x over previous
import dataclasses
import functools

import jax
import jax.numpy as jnp
import numpy as np
from jax import lax
from jax.experimental import pallas as pl
from jax.experimental.pallas import tpu as pltpu
from jax.experimental.pallas import tpu_sc as plsc

DIM = 128
BATCH = 16384
NC = 2
NS = 16
NW = NC * NS
BPW = BATCH // NW
CHUNK = 64
NCHUNK = BPW // CHUNK
SLOTS = 4
NQUAD = NCHUNK // SLOTS
LANES = 16
NSEG = DIM // LANES

_BITREV = [0, 8, 4, 12, 2, 10, 6, 14, 1, 9, 5, 13, 3, 11, 7, 15]


def kernel(center_words, context_words, center_table, context_table):
    cw = center_words.astype(jnp.int32)
    xw = context_words.astype(jnp.int32)

    mesh = plsc.VectorSubcoreMesh(core_axis_name="c", subcore_axis_name="s")

    cp = pltpu.CompilerParams()
    if "needs_layout_passes" in pltpu.CompilerParams.__dataclass_fields__:
        cp = dataclasses.replace(cp, needs_layout_passes=False)

    @functools.partial(
        pl.kernel,
        compiler_params=cp,
        out_type=jax.ShapeDtypeStruct((BATCH,), jnp.float32),
        mesh=mesh,
        scratch_types=[
            pltpu.VMEM((BPW,), jnp.int32),
            pltpu.VMEM((BPW,), jnp.int32),
            pltpu.VMEM((SLOTS, CHUNK, DIM), jnp.float32),
            pltpu.VMEM((SLOTS, CHUNK, DIM), jnp.float32),
            pltpu.VMEM((BPW,), jnp.float32),
            pltpu.VMEM((CHUNK, LANES), jnp.float32),
        ] + [pltpu.SemaphoreType.DMA] * (2 * SLOTS + 1),
    )
    def skipgram(cw_hbm, xw_hbm, ct_hbm, xt_hbm, out_hbm,
                 cidx, xidx, crows, xrows, scores, accbuf, *dma_sems):
        wid = lax.axis_index("s") * NC + lax.axis_index("c")
        base = wid * BPW
        c0 = pltpu.make_async_copy(cw_hbm.at[pl.ds(base, CHUNK)],
                                   cidx.at[pl.ds(0, CHUNK)], dma_sems[0])
        x0 = pltpu.make_async_copy(xw_hbm.at[pl.ds(base, CHUNK)],
                                   xidx.at[pl.ds(0, CHUNK)], dma_sems[1])
        c0.start(); x0.start()
        REST = BPW - CHUNK
        c1 = pltpu.make_async_copy(cw_hbm.at[pl.ds(base + CHUNK, REST)],
                                   cidx.at[pl.ds(CHUNK, REST)], dma_sems[2])
        x1 = pltpu.make_async_copy(xw_hbm.at[pl.ds(base + CHUNK, REST)],
                                   xidx.at[pl.ds(CHUNK, REST)], dma_sems[3])
        c1.start(); x1.start()

        lane = lax.iota(jnp.int32, LANES)
        perms = {k: (lane + k) & (LANES - 1) for k in (1, 2, 4, 8, 12, 14, 15)}
        masks = {s: (lane & (s - 1)) < (s // 2) for s in (16, 8, 4, 2)}

        gather_dnums = lax.GatherDimensionNumbers(
            offset_dims=(), collapsed_slice_dims=(0,), start_index_map=(0,))

        def rot(x, k):
            return lax.gather(
                x, perms[k % LANES][:, None], gather_dnums, (1,),
                mode=lax.GatherScatterMode.PROMISE_IN_BOUNDS)

        def merge(a, b, s):
            k = s // 2
            return jnp.where(masks[s], a + rot(a, k), b + rot(b, -k))

        sems = tuple((dma_sems[2 * i], dma_sems[2 * i + 1])
                     for i in range(SLOTS))

        def issue(j, slot):
            sel = pl.ds(j * CHUNK, CHUNK)
            pltpu.async_copy(ct_hbm.at[cidx.at[sel]], crows.at[slot],
                             sems[slot][0])
            pltpu.async_copy(xt_hbm.at[xidx.at[sel]], xrows.at[slot],
                             sems[slot][1])

        def wait(slot):
            sel = pl.ds(0, CHUNK)
            pltpu.make_async_copy(ct_hbm.at[cidx.at[sel]], crows.at[slot],
                                  sems[slot][0]).wait()
            pltpu.make_async_copy(xt_hbm.at[xidx.at[sel]], xrows.at[slot],
                                  sems[slot][1]).wait()

        def compute(j, slot):
            @plsc.parallel_loop(0, CHUNK)
            def _(r):
                acc = (crows[slot, r, pl.ds(0, LANES)]
                       * xrows[slot, r, pl.ds(0, LANES)])
                for t in range(1, NSEG):
                    acc = acc + (crows[slot, r, pl.ds(t * LANES, LANES)]
                                 * xrows[slot, r, pl.ds(t * LANES, LANES)])
                accbuf[r, pl.ds(0, LANES)] = acc

            @plsc.parallel_loop(0, CHUNK // LANES)
            def _(g):
                stack = []
                for leaf in range(LANES):
                    r = g * LANES + _BITREV[leaf]
                    node = (16, accbuf[r, pl.ds(0, LANES)])
                    while stack and stack[-1][0] == node[0]:
                        s, a = stack.pop()
                        node = (s // 2, merge(a, node[1], s))
                    stack.append(node)
                scores[pl.ds(j * CHUNK + g * LANES, LANES)] = stack[0][1]

            pltpu.async_copy(scores.at[pl.ds(j * CHUNK, CHUNK)],
                             out_hbm.at[pl.ds(wid * BPW + j * CHUNK, CHUNK)],
                             dma_sems[2 * SLOTS])

        c0.wait(); x0.wait()
        issue(0, 0)
        c1.wait(); x1.wait()
        for s0 in range(1, SLOTS):
            issue(s0, s0)

        @pl.loop(0, NQUAD)
        def _(p):
            for sl in range(SLOTS):
                jj = SLOTS * p + sl
                wait(sl)
                compute(jj, sl)
                nxt = jj + SLOTS

                @pl.when(nxt < NCHUNK)
                def _():
                    issue(nxt, sl)

        for _j in range(NCHUNK):
            pltpu.make_async_copy(
                scores.at[pl.ds(0, CHUNK)],
                out_hbm.at[pl.ds(wid * BPW, CHUNK)],
                dma_sems[2 * SLOTS]).wait()

    return skipgram(cw, xw, center_table, context_table)


# speedup vs baseline: 1.0853x; 1.0150x over previous
import dataclasses
import functools

import jax
import jax.numpy as jnp
import numpy as np
from jax import lax
from jax.experimental import pallas as pl
from jax.experimental.pallas import tpu as pltpu
from jax.experimental.pallas import tpu_sc as plsc

DIM = 128
BATCH = 16384
NC = 2
NS = 16
NW = NC * NS
BPW = BATCH // NW
CHUNK = 32
NCHUNK = BPW // CHUNK
SLOTS = 4
NQUAD = NCHUNK // SLOTS
LANES = 16
NSEG = DIM // LANES

_BITREV = [0, 8, 4, 12, 2, 10, 6, 14, 1, 9, 5, 13, 3, 11, 7, 15]


def kernel(center_words, context_words, center_table, context_table):
    cw = center_words.astype(jnp.int32)
    xw = context_words.astype(jnp.int32)

    mesh = plsc.VectorSubcoreMesh(core_axis_name="c", subcore_axis_name="s")

    cp = pltpu.CompilerParams()
    if "needs_layout_passes" in pltpu.CompilerParams.__dataclass_fields__:
        cp = dataclasses.replace(cp, needs_layout_passes=False)

    @functools.partial(
        pl.kernel,
        compiler_params=cp,
        out_type=jax.ShapeDtypeStruct((BATCH,), jnp.float32),
        mesh=mesh,
        scratch_types=[
            pltpu.VMEM((BPW,), jnp.int32),
            pltpu.VMEM((BPW,), jnp.int32),
            pltpu.VMEM((SLOTS, CHUNK, DIM), jnp.float32),
            pltpu.VMEM((SLOTS, CHUNK, DIM), jnp.float32),
            pltpu.VMEM((BPW,), jnp.float32),
            pltpu.VMEM((CHUNK, LANES), jnp.float32),
        ] + [pltpu.SemaphoreType.DMA] * (2 * SLOTS + 1),
    )
    def skipgram(cw_hbm, xw_hbm, ct_hbm, xt_hbm, out_hbm,
                 cidx, xidx, crows, xrows, scores, accbuf, *dma_sems):
        wid = lax.axis_index("s") * NC + lax.axis_index("c")
        base = wid * BPW
        c0 = pltpu.make_async_copy(cw_hbm.at[pl.ds(base, CHUNK)],
                                   cidx.at[pl.ds(0, CHUNK)], dma_sems[0])
        x0 = pltpu.make_async_copy(xw_hbm.at[pl.ds(base, CHUNK)],
                                   xidx.at[pl.ds(0, CHUNK)], dma_sems[1])
        c0.start(); x0.start()
        REST = BPW - CHUNK
        c1 = pltpu.make_async_copy(cw_hbm.at[pl.ds(base + CHUNK, REST)],
                                   cidx.at[pl.ds(CHUNK, REST)], dma_sems[2])
        x1 = pltpu.make_async_copy(xw_hbm.at[pl.ds(base + CHUNK, REST)],
                                   xidx.at[pl.ds(CHUNK, REST)], dma_sems[3])
        c1.start(); x1.start()

        lane = lax.iota(jnp.int32, LANES)
        perms = {k: (lane + k) & (LANES - 1) for k in (1, 2, 4, 8, 12, 14, 15)}
        masks = {s: (lane & (s - 1)) < (s // 2) for s in (16, 8, 4, 2)}

        gather_dnums = lax.GatherDimensionNumbers(
            offset_dims=(), collapsed_slice_dims=(0,), start_index_map=(0,))

        def rot(x, k):
            return lax.gather(
                x, perms[k % LANES][:, None], gather_dnums, (1,),
                mode=lax.GatherScatterMode.PROMISE_IN_BOUNDS)

        def merge(a, b, s):
            k = s // 2
            return jnp.where(masks[s], a + rot(a, k), b + rot(b, -k))

        sems = tuple((dma_sems[2 * i], dma_sems[2 * i + 1])
                     for i in range(SLOTS))

        def issue(j, slot):
            sel = pl.ds(j * CHUNK, CHUNK)
            pltpu.async_copy(ct_hbm.at[cidx.at[sel]], crows.at[slot],
                             sems[slot][0])
            pltpu.async_copy(xt_hbm.at[xidx.at[sel]], xrows.at[slot],
                             sems[slot][1])

        def wait(slot):
            sel = pl.ds(0, CHUNK)
            pltpu.make_async_copy(ct_hbm.at[cidx.at[sel]], crows.at[slot],
                                  sems[slot][0]).wait()
            pltpu.make_async_copy(xt_hbm.at[xidx.at[sel]], xrows.at[slot],
                                  sems[slot][1]).wait()

        def compute(j, slot):
            @plsc.parallel_loop(0, CHUNK)
            def _(r):
                acc = (crows[slot, r, pl.ds(0, LANES)]
                       * xrows[slot, r, pl.ds(0, LANES)])
                for t in range(1, NSEG):
                    acc = acc + (crows[slot, r, pl.ds(t * LANES, LANES)]
                                 * xrows[slot, r, pl.ds(t * LANES, LANES)])
                accbuf[r, pl.ds(0, LANES)] = acc

            @plsc.parallel_loop(0, CHUNK // LANES)
            def _(g):
                stack = []
                for leaf in range(LANES):
                    r = g * LANES + _BITREV[leaf]
                    node = (16, accbuf[r, pl.ds(0, LANES)])
                    while stack and stack[-1][0] == node[0]:
                        s, a = stack.pop()
                        node = (s // 2, merge(a, node[1], s))
                    stack.append(node)
                scores[pl.ds(j * CHUNK + g * LANES, LANES)] = stack[0][1]

            pltpu.async_copy(scores.at[pl.ds(j * CHUNK, CHUNK)],
                             out_hbm.at[pl.ds(wid * BPW + j * CHUNK, CHUNK)],
                             dma_sems[2 * SLOTS])

        c0.wait(); x0.wait()
        issue(0, 0)
        c1.wait(); x1.wait()
        for s0 in range(1, SLOTS):
            issue(s0, s0)

        @pl.loop(0, NQUAD)
        def _(p):
            for sl in range(SLOTS):
                jj = SLOTS * p + sl
                wait(sl)
                compute(jj, sl)
                nxt = jj + SLOTS

                @pl.when(nxt < NCHUNK)
                def _():
                    issue(nxt, sl)

        for _j in range(NCHUNK):
            pltpu.make_async_copy(
                scores.at[pl.ds(0, CHUNK)],
                out_hbm.at[pl.ds(wid * BPW, CHUNK)],
                dma_sems[2 * SLOTS]).wait()

    return skipgram(cw, xw, center_table, context_table)
